# Initial kernel scaffold; baseline (speedup 1.0000x reference)
import jax
import jax.numpy as jnp
from jax.experimental import pallas as pl


def kernel(node_feature, adj, w0, a_src0, a_dst0, b0, w1, a_src1, a_dst1, b1):
    raise NotImplementedError("write your pallas kernel here")


# speedup vs baseline: 1.5699x; 1.5699x over previous
import functools

import jax
import jax.numpy as jnp
from jax.experimental import pallas as pl
from jax.experimental.pallas import tpu as pltpu

LEAKY_ALPHA = 0.2
MASK_VAL = -999.0


def _proj_kernel(x_ref, w_ref, asrc_ref, adst_ref, h_ref, ss_ref, sd_ref):
    x = x_ref[0]
    w = w_ref[0]
    h = jnp.dot(x, w, preferred_element_type=jnp.float32)
    t = jnp.tanh(h)
    ss = jax.lax.dot_general(t, asrc_ref[0], (((1,), (0,)), ((), ())),
                             preferred_element_type=jnp.float32)
    sd = jax.lax.dot_general(adst_ref[0], t, (((0,), (1,)), ((), ())),
                             preferred_element_type=jnp.float32)
    h_ref[0, 0] = h
    ss_ref[0, 0] = ss
    sd_ref[0, 0] = sd


def _attn_kernel(ss_ref, sd_ref, adj_ref, h_ref, b_ref, x_ref, w_ref=None,
                 *, write_weight):
    head = pl.program_id(2)
    s = ss_ref[0, 0] + sd_ref[0, 0]
    s = jnp.where(s > 0, s, LEAKY_ALPHA * s)
    s = jnp.where(adj_ref[0] == 0.0, MASK_VAL, s)
    m = jnp.max(s, axis=1, keepdims=True)
    e = jnp.exp(s - m)
    a = e / jnp.sum(e, axis=1, keepdims=True)
    h = h_ref[0, head]
    x_ref[0, 0] = jnp.dot(a, h, preferred_element_type=jnp.float32) + b_ref[0]
    if write_weight:
        w_ref[0, 0] = a


def _gat_layer(x, adj, w, a_src, a_dst, b, *, block_rows, write_weight):
    B, N, in_dim = x.shape
    H, _, D = w.shape
    R = N // block_rows

    h, ss, sd = pl.pallas_call(
        _proj_kernel,
        grid=(B, H),
        in_specs=[
            pl.BlockSpec((1, N, in_dim), lambda bb, hh: (bb, 0, 0)),
            pl.BlockSpec((1, in_dim, D), lambda bb, hh: (hh, 0, 0)),
            pl.BlockSpec((1, D, 1), lambda bb, hh: (hh, 0, 0)),
            pl.BlockSpec((1, D, 1), lambda bb, hh: (hh, 0, 0)),
        ],
        out_specs=[
            pl.BlockSpec((1, 1, N, D), lambda bb, hh: (bb, hh, 0, 0)),
            pl.BlockSpec((1, 1, N, 1), lambda bb, hh: (bb, hh, 0, 0)),
            pl.BlockSpec((1, 1, 1, N), lambda bb, hh: (bb, hh, 0, 0)),
        ],
        out_shape=[
            jax.ShapeDtypeStruct((B, H, N, D), jnp.float32),
            jax.ShapeDtypeStruct((B, H, N, 1), jnp.float32),
            jax.ShapeDtypeStruct((B, H, 1, N), jnp.float32),
        ],
    )(x, w, a_src, a_dst)

    bias = b.reshape(1, D)
    out_shapes = [jax.ShapeDtypeStruct((B, H, N, D), jnp.float32)]
    out_specs = [pl.BlockSpec((1, 1, block_rows, D),
                              lambda bb, rr, hh: (bb, hh, rr, 0))]
    if write_weight:
        out_shapes.append(jax.ShapeDtypeStruct((B, H, N, N), jnp.float32))
        out_specs.append(pl.BlockSpec((1, 1, block_rows, N),
                                      lambda bb, rr, hh: (bb, hh, rr, 0)))

    outs = pl.pallas_call(
        functools.partial(_attn_kernel, write_weight=write_weight),
        grid=(B, R, H),
        in_specs=[
            pl.BlockSpec((1, 1, block_rows, 1), lambda bb, rr, hh: (bb, hh, rr, 0)),
            pl.BlockSpec((1, 1, 1, N), lambda bb, rr, hh: (bb, hh, 0, 0)),
            pl.BlockSpec((1, block_rows, N), lambda bb, rr, hh: (bb, rr, 0)),
            pl.BlockSpec((1, H, N, D), lambda bb, rr, hh: (bb, 0, 0, 0)),
            pl.BlockSpec((1, D), lambda bb, rr, hh: (0, 0)),
        ],
        out_specs=out_specs,
        out_shape=out_shapes,
    )(ss, sd, adj, h, bias)

    if write_weight:
        x_out, weight = outs
    else:
        (x_out,), weight = outs, None
    return jnp.transpose(x_out, (0, 2, 1, 3)).reshape(B, N, H * D), weight


def kernel(node_feature, adj, w0, a_src0, a_dst0, b0, w1, a_src1, a_dst1, b1):
    x, _ = _gat_layer(node_feature, adj, w0, a_src0, a_dst0, b0,
                      block_rows=256, write_weight=False)
    x, weight = _gat_layer(x, adj, w1, a_src1, a_dst1, b1,
                           block_rows=256, write_weight=True)
    return x, weight


# speedup vs baseline: 1.8564x; 1.1825x over previous
import functools

import jax
import jax.numpy as jnp
from jax.experimental import pallas as pl
from jax.experimental.pallas import tpu as pltpu

LEAKY_ALPHA = 0.2
MASK_VAL = -999.0


def _proj_kernel(x_ref, w_ref, asrc_ref, adst_ref, h_ref, ss_ref, sd_ref):
    x = x_ref[0]
    w = w_ref[0]
    h = jnp.dot(x, w, preferred_element_type=jnp.float32)
    t = jnp.tanh(h)
    ss = jax.lax.dot_general(t, asrc_ref[0], (((1,), (0,)), ((), ())),
                             preferred_element_type=jnp.float32)
    sd = jax.lax.dot_general(adst_ref[0], t, (((0,), (1,)), ((), ())),
                             preferred_element_type=jnp.float32)
    h_ref[0, 0] = h
    ss_ref[0, 0] = ss
    sd_ref[0, 0] = sd


def _attn_kernel(ss_ref, sd_ref, adj_ref, h_ref, b_ref, x_ref, w_ref=None,
                 *, write_weight):
    head = pl.program_id(2)
    ss = ss_ref[0, 0]
    sd = sd_ref[0, 0]
    m = ss + jnp.max(sd)
    m = jnp.where(m > 0, m, LEAKY_ALPHA * m)
    s = ss + sd
    s = jnp.maximum(s, LEAKY_ALPHA * s)
    e = jnp.exp(s - m) * adj_ref[0]
    denom = jnp.sum(e, axis=1, keepdims=True)
    zero_row = denom == 0.0
    r = 1.0 / jnp.where(zero_row, 1.0, denom)
    u = jnp.where(zero_row, 1.0 / e.shape[1], 0.0)
    a = e * r + u
    h = h_ref[0, head]
    x_ref[0, 0] = jnp.dot(a, h, preferred_element_type=jnp.float32) + b_ref[0]
    if write_weight:
        w_ref[0, 0] = a


def _gat_layer(x, adj, w, a_src, a_dst, b, *, block_rows, write_weight):
    B, N, in_dim = x.shape
    H, _, D = w.shape
    R = N // block_rows

    h, ss, sd = pl.pallas_call(
        _proj_kernel,
        grid=(B, H),
        in_specs=[
            pl.BlockSpec((1, N, in_dim), lambda bb, hh: (bb, 0, 0)),
            pl.BlockSpec((1, in_dim, D), lambda bb, hh: (hh, 0, 0)),
            pl.BlockSpec((1, D, 1), lambda bb, hh: (hh, 0, 0)),
            pl.BlockSpec((1, D, 1), lambda bb, hh: (hh, 0, 0)),
        ],
        out_specs=[
            pl.BlockSpec((1, 1, N, D), lambda bb, hh: (bb, hh, 0, 0)),
            pl.BlockSpec((1, 1, N, 1), lambda bb, hh: (bb, hh, 0, 0)),
            pl.BlockSpec((1, 1, 1, N), lambda bb, hh: (bb, hh, 0, 0)),
        ],
        out_shape=[
            jax.ShapeDtypeStruct((B, H, N, D), jnp.float32),
            jax.ShapeDtypeStruct((B, H, N, 1), jnp.float32),
            jax.ShapeDtypeStruct((B, H, 1, N), jnp.float32),
        ],
    )(x, w, a_src, a_dst)

    bias = b.reshape(1, D)
    out_shapes = [jax.ShapeDtypeStruct((B, H, N, D), jnp.float32)]
    out_specs = [pl.BlockSpec((1, 1, block_rows, D),
                              lambda bb, rr, hh: (bb, hh, rr, 0))]
    if write_weight:
        out_shapes.append(jax.ShapeDtypeStruct((B, H, N, N), jnp.float32))
        out_specs.append(pl.BlockSpec((1, 1, block_rows, N),
                                      lambda bb, rr, hh: (bb, hh, rr, 0)))

    outs = pl.pallas_call(
        functools.partial(_attn_kernel, write_weight=write_weight),
        grid=(B, R, H),
        in_specs=[
            pl.BlockSpec((1, 1, block_rows, 1), lambda bb, rr, hh: (bb, hh, rr, 0)),
            pl.BlockSpec((1, 1, 1, N), lambda bb, rr, hh: (bb, hh, 0, 0)),
            pl.BlockSpec((1, block_rows, N), lambda bb, rr, hh: (bb, rr, 0)),
            pl.BlockSpec((1, H, N, D), lambda bb, rr, hh: (bb, 0, 0, 0)),
            pl.BlockSpec((1, D), lambda bb, rr, hh: (0, 0)),
        ],
        out_specs=out_specs,
        out_shape=out_shapes,
    )(ss, sd, adj, h, bias)

    if write_weight:
        x_out, weight = outs
    else:
        (x_out,), weight = outs, None
    return jnp.transpose(x_out, (0, 2, 1, 3)).reshape(B, N, H * D), weight


def kernel(node_feature, adj, w0, a_src0, a_dst0, b0, w1, a_src1, a_dst1, b1):
    x, _ = _gat_layer(node_feature, adj, w0, a_src0, a_dst0, b0,
                      block_rows=512, write_weight=False)
    x, weight = _gat_layer(x, adj, w1, a_src1, a_dst1, b1,
                           block_rows=512, write_weight=True)
    return x, weight


# speedup vs baseline: 1.9223x; 1.0355x over previous
import functools

import jax
import jax.numpy as jnp
from jax.experimental import pallas as pl
from jax.experimental.pallas import tpu as pltpu

LEAKY_ALPHA = 0.2
MASK_VAL = -999.0


def _proj_kernel(x_ref, w_ref, asrc_ref, adst_ref, h_ref, ss_ref, sd_ref):
    x = x_ref[0]
    w = w_ref[0]
    h = jnp.dot(x, w, preferred_element_type=jnp.float32)
    t = jnp.tanh(h)
    ss = jax.lax.dot_general(t, asrc_ref[0], (((1,), (0,)), ((), ())),
                             preferred_element_type=jnp.float32)
    sd = jax.lax.dot_general(adst_ref[0], t, (((0,), (1,)), ((), ())),
                             preferred_element_type=jnp.float32)
    h_ref[0, 0] = h
    ss_ref[0, 0] = ss
    sd_ref[0, 0] = sd


def _attn_kernel(ss_ref, sd_ref, adj_ref, h_ref, b_ref, x_ref, w_ref=None,
                 *, write_weight):
    head = pl.program_id(2)
    ss = ss_ref[0, 0]
    sd = sd_ref[0, 0]
    m = ss + jnp.max(sd)
    m = jnp.where(m > 0, m, LEAKY_ALPHA * m)
    p = ss - m
    q = LEAKY_ALPHA * ss - m
    e = jnp.exp(jnp.maximum(p + sd, q + LEAKY_ALPHA * sd)) * adj_ref[0]
    denom = jnp.sum(e, axis=1, keepdims=True)
    zero_row = denom == 0.0
    r = 1.0 / jnp.where(zero_row, 1.0, denom)
    u = jnp.where(zero_row, 1.0 / e.shape[1], 0.0)
    a = e * r + u
    h = h_ref[0, head]
    x_ref[0, 0] = jnp.dot(a, h, preferred_element_type=jnp.float32) + b_ref[0]
    if write_weight:
        w_ref[0, 0] = a


def _gat_layer(x, adj, w, a_src, a_dst, b, *, block_rows, write_weight):
    B, N, in_dim = x.shape
    H, _, D = w.shape
    R = N // block_rows

    h, ss, sd = pl.pallas_call(
        _proj_kernel,
        grid=(B, H),
        in_specs=[
            pl.BlockSpec((1, N, in_dim), lambda bb, hh: (bb, 0, 0)),
            pl.BlockSpec((1, in_dim, D), lambda bb, hh: (hh, 0, 0)),
            pl.BlockSpec((1, D, 1), lambda bb, hh: (hh, 0, 0)),
            pl.BlockSpec((1, D, 1), lambda bb, hh: (hh, 0, 0)),
        ],
        out_specs=[
            pl.BlockSpec((1, 1, N, D), lambda bb, hh: (bb, hh, 0, 0)),
            pl.BlockSpec((1, 1, N, 1), lambda bb, hh: (bb, hh, 0, 0)),
            pl.BlockSpec((1, 1, 1, N), lambda bb, hh: (bb, hh, 0, 0)),
        ],
        out_shape=[
            jax.ShapeDtypeStruct((B, H, N, D), jnp.float32),
            jax.ShapeDtypeStruct((B, H, N, 1), jnp.float32),
            jax.ShapeDtypeStruct((B, H, 1, N), jnp.float32),
        ],
    )(x, w, a_src, a_dst)

    bias = b.reshape(1, D)
    out_shapes = [jax.ShapeDtypeStruct((B, H, N, D), jnp.float32)]
    out_specs = [pl.BlockSpec((1, 1, block_rows, D),
                              lambda bb, rr, hh: (bb, hh, rr, 0))]
    if write_weight:
        out_shapes.append(jax.ShapeDtypeStruct((B, H, N, N), jnp.float32))
        out_specs.append(pl.BlockSpec((1, 1, block_rows, N),
                                      lambda bb, rr, hh: (bb, hh, rr, 0)))

    outs = pl.pallas_call(
        functools.partial(_attn_kernel, write_weight=write_weight),
        grid=(B, R, H),
        in_specs=[
            pl.BlockSpec((1, 1, block_rows, 1), lambda bb, rr, hh: (bb, hh, rr, 0)),
            pl.BlockSpec((1, 1, 1, N), lambda bb, rr, hh: (bb, hh, 0, 0)),
            pl.BlockSpec((1, block_rows, N), lambda bb, rr, hh: (bb, rr, 0)),
            pl.BlockSpec((1, H, N, D), lambda bb, rr, hh: (bb, 0, 0, 0)),
            pl.BlockSpec((1, D), lambda bb, rr, hh: (0, 0)),
        ],
        out_specs=out_specs,
        out_shape=out_shapes,
    )(ss, sd, adj, h, bias)

    if write_weight:
        x_out, weight = outs
    else:
        (x_out,), weight = outs, None
    return jnp.transpose(x_out, (0, 2, 1, 3)).reshape(B, N, H * D), weight


def kernel(node_feature, adj, w0, a_src0, a_dst0, b0, w1, a_src1, a_dst1, b1):
    x, _ = _gat_layer(node_feature, adj, w0, a_src0, a_dst0, b0,
                      block_rows=512, write_weight=False)
    x, weight = _gat_layer(x, adj, w1, a_src1, a_dst1, b1,
                           block_rows=512, write_weight=True)
    return x, weight


# speedup vs baseline: 2.7164x; 1.4131x over previous
import functools

import jax
import jax.numpy as jnp
from jax.experimental import pallas as pl
from jax.experimental.pallas import tpu as pltpu

LEAKY_ALPHA = 0.2


def _proj_kernel(x_ref, w_ref, asrc_ref, adst_ref, h_ref, ss_ref, sd_ref):
    x = x_ref[0]
    w = w_ref[0]
    h = jnp.dot(x, w, preferred_element_type=jnp.float32)
    t = jnp.tanh(h)
    ss = jax.lax.dot_general(t, asrc_ref[0], (((1,), (0,)), ((), ())),
                             preferred_element_type=jnp.float32)
    sd = jax.lax.dot_general(adst_ref[0], t, (((0,), (1,)), ((), ())),
                             preferred_element_type=jnp.float32)
    h_ref[0, 0] = h
    ss_ref[0, 0] = ss
    sd_ref[0, 0] = sd


def _attn_kernel(ss_ref, sd_ref, adj_ref, h_ref, b_ref, x_ref, w_ref=None,
                 *, n_heads, d_out, write_weight):
    adj = adj_ref[0]
    for head in range(n_heads):
        ss = ss_ref[0, head]
        sd = sd_ref[0, head]
        m = ss + jnp.max(sd)
        m = jnp.where(m > 0, m, LEAKY_ALPHA * m)
        p = ss - m
        q = LEAKY_ALPHA * ss - m
        e = jnp.exp(jnp.maximum(p + sd, q + LEAKY_ALPHA * sd)) * adj
        denom = jnp.sum(e, axis=1, keepdims=True)
        zero_row = denom == 0.0
        r = 1.0 / jnp.where(zero_row, 1.0, denom)
        u = jnp.where(zero_row, 1.0 / e.shape[1], 0.0)
        a = e * r + u
        h = h_ref[0, head]
        x_ref[0, :, head * d_out:(head + 1) * d_out] = (
            jnp.dot(a, h, preferred_element_type=jnp.float32) + b_ref[0])
        if write_weight:
            w_ref[0, head] = a


def _gat_layer(x, adj, w, a_src, a_dst, b, *, block_rows, write_weight):
    B, N, in_dim = x.shape
    H, _, D = w.shape
    R = N // block_rows

    h, ss, sd = pl.pallas_call(
        _proj_kernel,
        grid=(B, H),
        in_specs=[
            pl.BlockSpec((1, N, in_dim), lambda bb, hh: (bb, 0, 0)),
            pl.BlockSpec((1, in_dim, D), lambda bb, hh: (hh, 0, 0)),
            pl.BlockSpec((1, D, 1), lambda bb, hh: (hh, 0, 0)),
            pl.BlockSpec((1, D, 1), lambda bb, hh: (hh, 0, 0)),
        ],
        out_specs=[
            pl.BlockSpec((1, 1, N, D), lambda bb, hh: (bb, hh, 0, 0)),
            pl.BlockSpec((1, 1, N, 1), lambda bb, hh: (bb, hh, 0, 0)),
            pl.BlockSpec((1, 1, 1, N), lambda bb, hh: (bb, hh, 0, 0)),
        ],
        out_shape=[
            jax.ShapeDtypeStruct((B, H, N, D), jnp.float32),
            jax.ShapeDtypeStruct((B, H, N, 1), jnp.float32),
            jax.ShapeDtypeStruct((B, H, 1, N), jnp.float32),
        ],
    )(x, w, a_src, a_dst)

    bias = b.reshape(1, D)
    out_shapes = [jax.ShapeDtypeStruct((B, N, H * D), jnp.float32)]
    out_specs = [pl.BlockSpec((1, block_rows, H * D),
                              lambda bb, rr: (bb, rr, 0))]
    if write_weight:
        out_shapes.append(jax.ShapeDtypeStruct((B, H, N, N), jnp.float32))
        out_specs.append(pl.BlockSpec((1, H, block_rows, N),
                                      lambda bb, rr: (bb, 0, rr, 0)))

    outs = pl.pallas_call(
        functools.partial(_attn_kernel, n_heads=H, d_out=D,
                          write_weight=write_weight),
        grid=(B, R),
        in_specs=[
            pl.BlockSpec((1, H, block_rows, 1), lambda bb, rr: (bb, 0, rr, 0)),
            pl.BlockSpec((1, H, 1, N), lambda bb, rr: (bb, 0, 0, 0)),
            pl.BlockSpec((1, block_rows, N), lambda bb, rr: (bb, rr, 0)),
            pl.BlockSpec((1, H, N, D), lambda bb, rr: (bb, 0, 0, 0)),
            pl.BlockSpec((1, D), lambda bb, rr: (0, 0)),
        ],
        out_specs=out_specs,
        out_shape=out_shapes,
    )(ss, sd, adj, h, bias)

    if write_weight:
        x_out, weight = outs
    else:
        (x_out,), weight = outs, None
    return x_out, weight


def kernel(node_feature, adj, w0, a_src0, a_dst0, b0, w1, a_src1, a_dst1, b1):
    x, _ = _gat_layer(node_feature, adj, w0, a_src0, a_dst0, b0,
                      block_rows=512, write_weight=False)
    x, weight = _gat_layer(x, adj, w1, a_src1, a_dst1, b1,
                           block_rows=256, write_weight=True)
    return x, weight
